# baseline (speedup 1.0000x reference)
import functools

import jax
import jax.numpy as jnp
from jax import lax
from jax.experimental import pallas as pl
from jax.experimental.pallas import tpu as pltpu
from jax.experimental.pallas import tpu_sc as plsc

_R = 512


_LN = 128


def _embed_gather(x_flat, table):
    n_tok = x_flat.shape[0]
    d = table.shape[1]
    seg = d // _LN
    idx2 = (x_flat.reshape(-1, 8)[:, None, :] * seg
            + jnp.arange(seg, dtype=jnp.int32)[None, :, None]).reshape(-1)
    table2 = table.reshape(-1, _LN)

    info = plsc.get_sparse_core_info()
    nc, ns = info.num_cores, info.num_subcores
    nw = nc * ns
    rows_out = n_tok * seg
    per_w = rows_out // nw
    k6 = per_w // _LN

    mesh = plsc.VectorSubcoreMesh(core_axis_name="c", subcore_axis_name="s")

    @functools.partial(
        pl.kernel,
        mesh=mesh,
        out_type=jax.ShapeDtypeStruct((rows_out, _LN), jnp.float32),
        scratch_types=[
            pltpu.VMEM((k6, _LN), jnp.int32),
            pltpu.VMEM((per_w, _LN), jnp.float32),
            pltpu.SemaphoreType.DMA,
        ],
    )
    def gather_kernel(idx_hbm, table_hbm, out_hbm, idx_v, rows_v, sem):
        wid = lax.axis_index("s") * nc + lax.axis_index("c")
        base = wid * per_w
        for b in range(k6):
            pltpu.sync_copy(idx_hbm.at[pl.ds(base + b * _LN, _LN)],
                            idx_v.at[b])
        copies = [
            pltpu.async_copy(table_hbm.at[idx_v.at[b]],
                             rows_v.at[pl.ds(b * _LN, _LN)], sem)
            for b in range(k6)
        ]
        for cp in copies:
            cp.wait()
        pltpu.sync_copy(rows_v, out_hbm.at[pl.ds(base, per_w)])

    return gather_kernel(idx2, table2)


def _split3(w):
    hi = w.astype(jnp.bfloat16)
    lo = (w - hi.astype(jnp.float32)).astype(jnp.bfloat16)
    return hi, lo


def _cat3(w, axis=0):
    hi, lo = _split3(w)
    return jnp.concatenate([hi, lo, hi], axis=axis)


def _dot3(a, b_cat, dn):
    a_hi = a.astype(jnp.bfloat16)
    a_lo = (a - a_hi.astype(jnp.float32)).astype(jnp.bfloat16)
    a_cat = jnp.concatenate([a_hi, a_hi, a_lo], axis=1)
    return lax.dot_general(a_cat, b_cat, dn,
                           preferred_element_type=jnp.float32)


_NT = (((1,), (0,)), ((), ()))
_TT = (((1,), (1,)), ((), ()))


_T0 = 16


def _two_level_scan(x, tp, rr, dd):
    t1 = rr // _T0
    x4 = x.reshape(t1, _T0, dd)
    s = 1
    while s < _T0:
        dk = tp[s - 1:s, :].reshape(1, 1, dd)
        zpad = jnp.zeros((t1, s, dd), jnp.float32)
        x4 = x4 + dk * jnp.concatenate([zpad, x4[:, :_T0 - s, :]], axis=1)
        s *= 2
    blk = x4[:, _T0 - 1, :]
    inc = blk
    s = 1
    while s < t1:
        dk = tp[s * _T0 - 1:s * _T0, :]
        zpad = jnp.zeros((s, dd), jnp.float32)
        inc = inc + dk * jnp.concatenate([zpad, inc[:t1 - s, :]], axis=0)
        s *= 2
    exc = jnp.concatenate([jnp.zeros((1, dd), jnp.float32), inc[:t1 - 1, :]],
                          axis=0)
    dsub = tp[:_T0, :].reshape(1, _T0, dd)
    x4 = x4 + dsub * exc.reshape(t1, 1, dd)
    return x4.reshape(rr, dd)


def _mega_body(cpb, cap, h_ref, td_ref, wrkv_ref, wo_ref, mk_ref, mv_ref,
               wc_ref, wd_ref, wout_ref, bout_ref,
               out_ref, cn_ref, cd_ref, a_ref, b_ref):
    i = pl.program_id(0)
    seg = h_ref.shape[1]
    rr = h_ref.shape[0] * h_ref.shape[2]
    dd = seg * h_ref.shape[3]

    @pl.when(i % cpb == 0)
    def _():
        cn_ref[...] = jnp.zeros_like(cn_ref)
        cd_ref[...] = jnp.zeros_like(cd_ref)

    h4 = h_ref[...]
    h = jnp.concatenate(
        [h4[:, b, :, :].reshape(rr, _LN) for b in range(seg)], axis=1)
    decay = jnp.exp(-jnp.exp(td_ref[...]))

    rkv = lax.dot_general(h, wrkv_ref[...], _NT,
                          preferred_element_type=jnp.float32)
    r = jax.nn.sigmoid(rkv[:, :dd])
    k = rkv[:, dd:2 * dd]
    v = rkv[:, 2 * dd:]
    ek = jnp.exp(jnp.clip(k, -30.0, 30.0))

    a_ref[...] = ek * v
    b_ref[...] = ek

    def _step(t, _):
        st_n = decay * cn_ref[...] + a_ref[pl.ds(t, 1), :]
        st_d = decay * cd_ref[...] + b_ref[pl.ds(t, 1), :]
        cn_ref[...] = st_n
        cd_ref[...] = st_d
        a_ref[pl.ds(t, 1), :] = st_n
        b_ref[pl.ds(t, 1), :] = st_d
        return 0

    lax.fori_loop(0, rr, _step, 0)
    wkv = a_ref[...] / (b_ref[...] + 1e-6)

    h2 = h + lax.dot_general(r * wkv, wo_ref[...], _NT,
                             preferred_element_type=jnp.float32)

    q = lax.dot_general(h2, wc_ref[...], _NT,
                        preferred_element_type=jnp.float32)
    c_dim = q.shape[1]
    scores = lax.dot_general(q, mk_ref[...], _TT,
                             preferred_element_type=jnp.float32)
    scores = scores * jnp.float32(1.0 / (c_dim ** 0.5))

    col = lax.broadcasted_iota(jnp.int32, (rr, cap), 1)
    m1 = jnp.max(scores, axis=1, keepdims=True)
    i1 = jnp.min(jnp.where(scores == m1, col, cap), axis=1, keepdims=True)
    masked = jnp.where(col == i1, jnp.float32(-jnp.inf), scores)
    m2 = jnp.max(masked, axis=1, keepdims=True)
    i2 = jnp.min(jnp.where(masked == m2, col, cap), axis=1, keepdims=True)
    e2 = jnp.exp(m2 - m1)
    w1 = 1.0 / (1.0 + e2)
    w2 = e2 / (1.0 + e2)
    wsel = jnp.where(col == i1, w1, 0.0) + jnp.where(col == i2, w2, 0.0)

    read = jnp.dot(wsel, mv_ref[...], preferred_element_type=jnp.float32)
    h3 = h2 + jnp.dot(read.astype(jnp.bfloat16), wd_ref[...],
                      preferred_element_type=jnp.float32)
    outt = lax.dot_general(wout_ref[...], h3.astype(jnp.bfloat16),
                           (((1,), (1,)), ((), ())),
                           preferred_element_type=jnp.float32)
    out_ref[...] = (outt + bout_ref[...]).reshape(1, outt.shape[0],
                                                  outt.shape[1])


def _tc_forward(h4, d, td, wr, wk, wv, wo, mem_keys, mem_values, wc, wd,
                w_out, b_out, chunks_per_batch):
    seg = d // _LN
    n_tok = h4.shape[0] // seg
    cap, c = mem_keys.shape
    vocab = w_out.shape[1]
    n_chunks = n_tok // _R
    h4 = h4.reshape(n_tok // 8, seg, 8, _LN)

    fixed = lambda i: (0, 0)
    call = pl.pallas_call(
        functools.partial(_mega_body, chunks_per_batch, cap),
        grid=(n_chunks,),
        in_specs=[
            pl.BlockSpec((_R // 8, seg, 8, _LN), lambda i: (i, 0, 0, 0)),
            pl.BlockSpec((1, d), fixed),
            pl.BlockSpec((d, 3 * d), fixed),
            pl.BlockSpec((d, d), fixed),
            pl.BlockSpec((cap, c), fixed),
            pl.BlockSpec((cap, c), fixed),
            pl.BlockSpec((d, c), fixed),
            pl.BlockSpec((c, d), fixed),
            pl.BlockSpec((vocab, d), fixed),
            pl.BlockSpec((vocab, 1), fixed),
        ],
        out_specs=pl.BlockSpec(
            (1, vocab, _R),
            lambda i: (i // chunks_per_batch, 0, i % chunks_per_batch)),
        out_shape=jax.ShapeDtypeStruct(
            (n_tok // (chunks_per_batch * _R), vocab,
             chunks_per_batch * _R), jnp.float32),
        scratch_shapes=[
            pltpu.VMEM((1, d), jnp.float32),
            pltpu.VMEM((1, d), jnp.float32),
            pltpu.VMEM((_R, d), jnp.float32),
            pltpu.VMEM((_R, d), jnp.float32),
        ],
        compiler_params=pltpu.CompilerParams(
            dimension_semantics=("arbitrary",),
        ),
    )
    wrkv = jnp.concatenate([wr, wk, wv], axis=1)
    return call(h4, td, wrkv, wo, mem_keys, mem_values, wc,
                wd.astype(jnp.bfloat16), w_out.T.astype(jnp.bfloat16),
                b_out.reshape(-1, 1))


def kernel(x, embed_table, time_decay, Wr, Wk, Wv, Wo, mem_keys, mem_values,
           Wc, Wd, W_out, b_out):
    b, s = x.shape
    vocab = W_out.shape[1]
    d = embed_table.shape[1]
    x_flat = x.reshape(-1).astype(jnp.int32)
    h4 = _embed_gather(x_flat, embed_table)
    out = _tc_forward(h4, d, time_decay.reshape(1, -1), Wr, Wk, Wv, Wo,
                      mem_keys, mem_values, Wc, Wd, W_out,
                      b_out, chunks_per_batch=s // _R)
    return out.swapaxes(1, 2)


# speedup vs baseline: 1.1602x; 1.1602x over previous
import functools

import jax
import jax.numpy as jnp
from jax import lax
from jax.experimental import pallas as pl
from jax.experimental.pallas import tpu as pltpu
from jax.experimental.pallas import tpu_sc as plsc

_R = 512


_LN = 128


def _embed_gather(x_flat, table):
    n_tok = x_flat.shape[0]
    d = table.shape[1]
    seg = d // _LN
    idx2 = (x_flat.reshape(-1, 8)[:, None, :] * seg
            + jnp.arange(seg, dtype=jnp.int32)[None, :, None]).reshape(-1)
    table2 = table.reshape(-1, _LN)

    info = plsc.get_sparse_core_info()
    nc, ns = info.num_cores, info.num_subcores
    nw = nc * ns
    rows_out = n_tok * seg
    per_w = rows_out // nw
    k6 = per_w // _LN

    mesh = plsc.VectorSubcoreMesh(core_axis_name="c", subcore_axis_name="s")

    @functools.partial(
        pl.kernel,
        mesh=mesh,
        out_type=jax.ShapeDtypeStruct((rows_out, _LN), jnp.float32),
        scratch_types=[
            pltpu.VMEM((k6, _LN), jnp.int32),
            pltpu.VMEM((per_w, _LN), jnp.float32),
            pltpu.SemaphoreType.DMA,
        ],
    )
    def gather_kernel(idx_hbm, table_hbm, out_hbm, idx_v, rows_v, sem):
        wid = lax.axis_index("s") * nc + lax.axis_index("c")
        base = wid * per_w
        for b in range(k6):
            pltpu.sync_copy(idx_hbm.at[pl.ds(base + b * _LN, _LN)],
                            idx_v.at[b])
        copies = [
            pltpu.async_copy(table_hbm.at[idx_v.at[b]],
                             rows_v.at[pl.ds(b * _LN, _LN)], sem)
            for b in range(k6)
        ]
        for cp in copies:
            cp.wait()
        pltpu.sync_copy(rows_v, out_hbm.at[pl.ds(base, per_w)])

    return gather_kernel(idx2, table2)


def _cat3(w, axis=0):
    hi, lo = _split3(w)
    return jnp.concatenate([hi, lo, hi], axis=axis)


def _dot3(a, b_cat, dn):
    a_hi = a.astype(jnp.bfloat16)
    a_lo = (a - a_hi.astype(jnp.float32)).astype(jnp.bfloat16)
    a_cat = jnp.concatenate([a_hi, a_hi, a_lo], axis=1)
    return lax.dot_general(a_cat, b_cat, dn,
                           preferred_element_type=jnp.float32)


_NT = (((1,), (0,)), ((), ()))
_TT = (((1,), (1,)), ((), ()))


_T0 = 16


def _two_level_scan(x, tp, rr, dd):
    t1 = rr // _T0
    x4 = x.reshape(t1, _T0, dd)
    s = 1
    while s < _T0:
        dk = tp[s - 1:s, :].reshape(1, 1, dd)
        zpad = jnp.zeros((t1, s, dd), jnp.float32)
        x4 = x4 + dk * jnp.concatenate([zpad, x4[:, :_T0 - s, :]], axis=1)
        s *= 2
    blk = x4[:, _T0 - 1, :]
    inc = blk
    s = 1
    while s < t1:
        dk = tp[s * _T0 - 1:s * _T0, :]
        zpad = jnp.zeros((s, dd), jnp.float32)
        inc = inc + dk * jnp.concatenate([zpad, inc[:t1 - s, :]], axis=0)
        s *= 2
    exc = jnp.concatenate([jnp.zeros((1, dd), jnp.float32), inc[:t1 - 1, :]],
                          axis=0)
    dsub = tp[:_T0, :].reshape(1, _T0, dd)
    x4 = x4 + dsub * exc.reshape(t1, 1, dd)
    return x4.reshape(rr, dd)


def _mega_body(cpb, cap, h_ref, td_ref, wrkv_ref, wo_ref, mk_ref, mv_ref,
               wc_ref, wd_ref, wout_ref, bout_ref,
               out_ref, cn_ref, cd_ref, a_ref, b_ref):
    i = pl.program_id(0)
    seg = h_ref.shape[1]
    rr = h_ref.shape[0] * h_ref.shape[2]
    dd = seg * h_ref.shape[3]

    @pl.when(i % cpb == 0)
    def _():
        cn_ref[...] = jnp.zeros_like(cn_ref)
        cd_ref[...] = jnp.zeros_like(cd_ref)

    h4 = h_ref[...]
    h = jnp.concatenate(
        [h4[:, b, :, :].reshape(rr, _LN) for b in range(seg)], axis=1)
    decay = jnp.exp(-jnp.exp(td_ref[...]))

    rkv = lax.dot_general(h, wrkv_ref[...], _NT,
                          preferred_element_type=jnp.float32)
    r = jax.nn.sigmoid(rkv[:, :dd])
    k = rkv[:, dd:2 * dd]
    v = rkv[:, 2 * dd:]
    ek = jnp.exp(jnp.clip(k, -30.0, 30.0))

    a_ref[...] = ek * v
    b_ref[...] = ek

    unroll = 8

    def _step(t, carry):
        st_n, st_d = carry
        base = t * unroll
        for j in range(unroll):
            st_n = decay * st_n + a_ref[pl.ds(base + j, 1), :]
            st_d = decay * st_d + b_ref[pl.ds(base + j, 1), :]
            a_ref[pl.ds(base + j, 1), :] = st_n
            b_ref[pl.ds(base + j, 1), :] = st_d
        return st_n, st_d

    st = lax.fori_loop(0, rr // unroll, _step,
                       (cn_ref[...], cd_ref[...]))
    cn_ref[...] = st[0]
    cd_ref[...] = st[1]
    wkv = a_ref[...] / (b_ref[...] + 1e-6)

    h2 = h + lax.dot_general(r * wkv, wo_ref[...], _NT,
                             preferred_element_type=jnp.float32)

    q = lax.dot_general(h2, wc_ref[...], _NT,
                        preferred_element_type=jnp.float32)
    c_dim = q.shape[1]
    scores = lax.dot_general(q, mk_ref[...], _TT,
                             preferred_element_type=jnp.float32)
    scores = scores * jnp.float32(1.0 / (c_dim ** 0.5))

    col = lax.broadcasted_iota(jnp.int32, (rr, cap), 1)
    m1 = jnp.max(scores, axis=1, keepdims=True)
    i1 = jnp.min(jnp.where(scores == m1, col, cap), axis=1, keepdims=True)
    masked = jnp.where(col == i1, jnp.float32(-jnp.inf), scores)
    m2 = jnp.max(masked, axis=1, keepdims=True)
    i2 = jnp.min(jnp.where(masked == m2, col, cap), axis=1, keepdims=True)
    e2 = jnp.exp(m2 - m1)
    w1 = 1.0 / (1.0 + e2)
    w2 = e2 / (1.0 + e2)
    wsel = jnp.where(col == i1, w1, 0.0) + jnp.where(col == i2, w2, 0.0)

    read = jnp.dot(wsel, mv_ref[...], preferred_element_type=jnp.float32)
    h3 = h2 + jnp.dot(read.astype(jnp.bfloat16), wd_ref[...],
                      preferred_element_type=jnp.float32)
    outt = lax.dot_general(wout_ref[...], h3.astype(jnp.bfloat16),
                           (((1,), (1,)), ((), ())),
                           preferred_element_type=jnp.float32)
    out_ref[...] = (outt + bout_ref[...]).reshape(1, outt.shape[0],
                                                  outt.shape[1])


def _tc_forward(h4, d, td, wr, wk, wv, wo, mem_keys, mem_values, wc, wd,
                w_out, b_out, chunks_per_batch):
    seg = d // _LN
    n_tok = h4.shape[0] // seg
    cap, c = mem_keys.shape
    vocab = w_out.shape[1]
    n_chunks = n_tok // _R
    h4 = h4.reshape(n_tok // 8, seg, 8, _LN)

    fixed = lambda i: (0, 0)
    call = pl.pallas_call(
        functools.partial(_mega_body, chunks_per_batch, cap),
        grid=(n_chunks,),
        in_specs=[
            pl.BlockSpec((_R // 8, seg, 8, _LN), lambda i: (i, 0, 0, 0)),
            pl.BlockSpec((1, d), fixed),
            pl.BlockSpec((d, 3 * d), fixed),
            pl.BlockSpec((d, d), fixed),
            pl.BlockSpec((cap, c), fixed),
            pl.BlockSpec((cap, c), fixed),
            pl.BlockSpec((d, c), fixed),
            pl.BlockSpec((c, d), fixed),
            pl.BlockSpec((vocab, d), fixed),
            pl.BlockSpec((vocab, 1), fixed),
        ],
        out_specs=pl.BlockSpec(
            (1, vocab, _R),
            lambda i: (i // chunks_per_batch, 0, i % chunks_per_batch)),
        out_shape=jax.ShapeDtypeStruct(
            (n_tok // (chunks_per_batch * _R), vocab,
             chunks_per_batch * _R), jnp.float32),
        scratch_shapes=[
            pltpu.VMEM((1, d), jnp.float32),
            pltpu.VMEM((1, d), jnp.float32),
            pltpu.VMEM((_R, d), jnp.float32),
            pltpu.VMEM((_R, d), jnp.float32),
        ],
        compiler_params=pltpu.CompilerParams(
            dimension_semantics=("arbitrary",),
        ),
    )
    wrkv = jnp.concatenate([wr, wk, wv], axis=1)
    return call(h4, td, wrkv, wo, mem_keys, mem_values, wc,
                wd.astype(jnp.bfloat16), w_out.T.astype(jnp.bfloat16),
                b_out.reshape(-1, 1))


def kernel(x, embed_table, time_decay, Wr, Wk, Wv, Wo, mem_keys, mem_values,
           Wc, Wd, W_out, b_out):
    b, s = x.shape
    vocab = W_out.shape[1]
    d = embed_table.shape[1]
    x_flat = x.reshape(-1).astype(jnp.int32)
    h4 = _embed_gather(x_flat, embed_table)
    out = _tc_forward(h4, d, time_decay.reshape(1, -1), Wr, Wk, Wv, Wo,
                      mem_keys, mem_values, Wc, Wd, W_out,
                      b_out, chunks_per_batch=s // _R)
    return out.swapaxes(1, 2)


# speedup vs baseline: 1.1682x; 1.0068x over previous
import functools

import jax
import jax.numpy as jnp
from jax import lax
from jax.experimental import pallas as pl
from jax.experimental.pallas import tpu as pltpu
from jax.experimental.pallas import tpu_sc as plsc

_R = 512


_LN = 128


def _embed_gather(x_flat, table):
    n_tok = x_flat.shape[0]
    d = table.shape[1]
    seg = d // _LN
    idx2 = (x_flat.reshape(-1, 8)[:, None, :] * seg
            + jnp.arange(seg, dtype=jnp.int32)[None, :, None]).reshape(-1)
    table2 = table.reshape(-1, _LN)

    info = plsc.get_sparse_core_info()
    nc, ns = info.num_cores, info.num_subcores
    nw = nc * ns
    rows_out = n_tok * seg
    per_w = rows_out // nw
    k6 = per_w // _LN

    mesh = plsc.VectorSubcoreMesh(core_axis_name="c", subcore_axis_name="s")

    @functools.partial(
        pl.kernel,
        mesh=mesh,
        out_type=jax.ShapeDtypeStruct((rows_out, _LN), jnp.float32),
        scratch_types=[
            pltpu.VMEM((k6, _LN), jnp.int32),
            pltpu.VMEM((per_w, _LN), jnp.float32),
            pltpu.SemaphoreType.DMA,
        ],
    )
    def gather_kernel(idx_hbm, table_hbm, out_hbm, idx_v, rows_v, sem):
        wid = lax.axis_index("s") * nc + lax.axis_index("c")
        base = wid * per_w
        for b in range(k6):
            pltpu.sync_copy(idx_hbm.at[pl.ds(base + b * _LN, _LN)],
                            idx_v.at[b])
        copies = [
            pltpu.async_copy(table_hbm.at[idx_v.at[b]],
                             rows_v.at[pl.ds(b * _LN, _LN)], sem)
            for b in range(k6)
        ]
        for cp in copies:
            cp.wait()
        pltpu.sync_copy(rows_v, out_hbm.at[pl.ds(base, per_w)])

    return gather_kernel(idx2, table2)


def _cat3(w, axis=0):
    hi, lo = _split3(w)
    return jnp.concatenate([hi, lo, hi], axis=axis)


def _dot3(a, b_cat, dn):
    a_hi = a.astype(jnp.bfloat16)
    a_lo = (a - a_hi.astype(jnp.float32)).astype(jnp.bfloat16)
    a_cat = jnp.concatenate([a_hi, a_hi, a_lo], axis=1)
    return lax.dot_general(a_cat, b_cat, dn,
                           preferred_element_type=jnp.float32)


_NT = (((1,), (0,)), ((), ()))
_TT = (((1,), (1,)), ((), ()))


_T0 = 16


def _two_level_scan(x, tp, rr, dd):
    t1 = rr // _T0
    x4 = x.reshape(t1, _T0, dd)
    s = 1
    while s < _T0:
        dk = tp[s - 1:s, :].reshape(1, 1, dd)
        zpad = jnp.zeros((t1, s, dd), jnp.float32)
        x4 = x4 + dk * jnp.concatenate([zpad, x4[:, :_T0 - s, :]], axis=1)
        s *= 2
    blk = x4[:, _T0 - 1, :]
    inc = blk
    s = 1
    while s < t1:
        dk = tp[s * _T0 - 1:s * _T0, :]
        zpad = jnp.zeros((s, dd), jnp.float32)
        inc = inc + dk * jnp.concatenate([zpad, inc[:t1 - s, :]], axis=0)
        s *= 2
    exc = jnp.concatenate([jnp.zeros((1, dd), jnp.float32), inc[:t1 - 1, :]],
                          axis=0)
    dsub = tp[:_T0, :].reshape(1, _T0, dd)
    x4 = x4 + dsub * exc.reshape(t1, 1, dd)
    return x4.reshape(rr, dd)


def _mega_body(cpb, cap, h_ref, td_ref, wrkv_ref, wo_ref, mk_ref, mv_ref,
               wc_ref, wd_ref, wout_ref, bout_ref,
               out_ref, cn_ref, cd_ref, a_ref, b_ref):
    i = pl.program_id(0)
    seg = h_ref.shape[1]
    rr = h_ref.shape[0] * h_ref.shape[2]
    dd = seg * h_ref.shape[3]

    @pl.when(i % cpb == 0)
    def _():
        cn_ref[...] = jnp.zeros_like(cn_ref)
        cd_ref[...] = jnp.zeros_like(cd_ref)

    h4 = h_ref[...]
    h = jnp.concatenate(
        [h4[:, b, :, :].reshape(rr, _LN) for b in range(seg)], axis=1)
    decay = jnp.exp(-jnp.exp(td_ref[...]))

    rkv = lax.dot_general(h, wrkv_ref[...], _NT,
                          preferred_element_type=jnp.float32)
    r = jax.nn.sigmoid(rkv[:, :dd])
    k = rkv[:, dd:2 * dd]
    v = rkv[:, 2 * dd:]
    ek = jnp.exp(jnp.clip(k, -30.0, 30.0))

    a_ref[...] = ek * v
    b_ref[...] = ek

    unroll = 16

    def _step(t, carry):
        st_n, st_d = carry
        base = t * unroll
        for j in range(unroll):
            st_n = decay * st_n + a_ref[pl.ds(base + j, 1), :]
            st_d = decay * st_d + b_ref[pl.ds(base + j, 1), :]
            a_ref[pl.ds(base + j, 1), :] = st_n
            b_ref[pl.ds(base + j, 1), :] = st_d
        return st_n, st_d

    st = lax.fori_loop(0, rr // unroll, _step,
                       (cn_ref[...], cd_ref[...]))
    cn_ref[...] = st[0]
    cd_ref[...] = st[1]
    wkv = a_ref[...] / (b_ref[...] + 1e-6)

    h2 = h + lax.dot_general(r * wkv, wo_ref[...], _NT,
                             preferred_element_type=jnp.float32)

    q = lax.dot_general(h2, wc_ref[...], _NT,
                        preferred_element_type=jnp.float32)
    c_dim = q.shape[1]
    scores = lax.dot_general(q, mk_ref[...], _TT,
                             preferred_element_type=jnp.float32)
    scores = scores * jnp.float32(1.0 / (c_dim ** 0.5))

    col = lax.broadcasted_iota(jnp.int32, (rr, cap), 1)
    m1 = jnp.max(scores, axis=1, keepdims=True)
    i1 = jnp.min(jnp.where(scores == m1, col, cap), axis=1, keepdims=True)
    masked = jnp.where(col == i1, jnp.float32(-jnp.inf), scores)
    m2 = jnp.max(masked, axis=1, keepdims=True)
    i2 = jnp.min(jnp.where(masked == m2, col, cap), axis=1, keepdims=True)
    e2 = jnp.exp(m2 - m1)
    w1 = 1.0 / (1.0 + e2)
    w2 = e2 / (1.0 + e2)
    wsel = jnp.where(col == i1, w1, 0.0) + jnp.where(col == i2, w2, 0.0)

    read = jnp.dot(wsel, mv_ref[...], preferred_element_type=jnp.float32)
    h3 = h2 + jnp.dot(read.astype(jnp.bfloat16), wd_ref[...],
                      preferred_element_type=jnp.float32)
    outt = lax.dot_general(wout_ref[...], h3.astype(jnp.bfloat16),
                           (((1,), (1,)), ((), ())),
                           preferred_element_type=jnp.float32)
    out_ref[...] = (outt + bout_ref[...]).reshape(1, outt.shape[0],
                                                  outt.shape[1])


def _tc_forward(h4, d, td, wr, wk, wv, wo, mem_keys, mem_values, wc, wd,
                w_out, b_out, chunks_per_batch):
    seg = d // _LN
    n_tok = h4.shape[0] // seg
    cap, c = mem_keys.shape
    vocab = w_out.shape[1]
    n_chunks = n_tok // _R
    h4 = h4.reshape(n_tok // 8, seg, 8, _LN)

    fixed = lambda i: (0, 0)
    call = pl.pallas_call(
        functools.partial(_mega_body, chunks_per_batch, cap),
        grid=(n_chunks,),
        in_specs=[
            pl.BlockSpec((_R // 8, seg, 8, _LN), lambda i: (i, 0, 0, 0)),
            pl.BlockSpec((1, d), fixed),
            pl.BlockSpec((d, 3 * d), fixed),
            pl.BlockSpec((d, d), fixed),
            pl.BlockSpec((cap, c), fixed),
            pl.BlockSpec((cap, c), fixed),
            pl.BlockSpec((d, c), fixed),
            pl.BlockSpec((c, d), fixed),
            pl.BlockSpec((vocab, d), fixed),
            pl.BlockSpec((vocab, 1), fixed),
        ],
        out_specs=pl.BlockSpec(
            (1, vocab, _R),
            lambda i: (i // chunks_per_batch, 0, i % chunks_per_batch)),
        out_shape=jax.ShapeDtypeStruct(
            (n_tok // (chunks_per_batch * _R), vocab,
             chunks_per_batch * _R), jnp.float32),
        scratch_shapes=[
            pltpu.VMEM((1, d), jnp.float32),
            pltpu.VMEM((1, d), jnp.float32),
            pltpu.VMEM((_R, d), jnp.float32),
            pltpu.VMEM((_R, d), jnp.float32),
        ],
        compiler_params=pltpu.CompilerParams(
            dimension_semantics=("arbitrary",),
        ),
    )
    wrkv = jnp.concatenate([wr, wk, wv], axis=1)
    return call(h4, td, wrkv, wo, mem_keys, mem_values, wc,
                wd.astype(jnp.bfloat16), w_out.T.astype(jnp.bfloat16),
                b_out.reshape(-1, 1))


def kernel(x, embed_table, time_decay, Wr, Wk, Wv, Wo, mem_keys, mem_values,
           Wc, Wd, W_out, b_out):
    b, s = x.shape
    vocab = W_out.shape[1]
    d = embed_table.shape[1]
    x_flat = x.reshape(-1).astype(jnp.int32)
    h4 = _embed_gather(x_flat, embed_table)
    out = _tc_forward(h4, d, time_decay.reshape(1, -1), Wr, Wk, Wv, Wo,
                      mem_keys, mem_values, Wc, Wd, W_out,
                      b_out, chunks_per_batch=s // _R)
    return out.swapaxes(1, 2)


# speedup vs baseline: 1.2254x; 1.0490x over previous
import functools

import jax
import jax.numpy as jnp
from jax import lax
from jax.experimental import pallas as pl
from jax.experimental.pallas import tpu as pltpu
from jax.experimental.pallas import tpu_sc as plsc

_R = 1024


_LN = 128


def _embed_gather(x_flat, table):
    n_tok = x_flat.shape[0]
    d = table.shape[1]
    seg = d // _LN
    idx2 = (x_flat.reshape(-1, 8)[:, None, :] * seg
            + jnp.arange(seg, dtype=jnp.int32)[None, :, None]).reshape(-1)
    table2 = table.reshape(-1, _LN)

    info = plsc.get_sparse_core_info()
    nc, ns = info.num_cores, info.num_subcores
    nw = nc * ns
    rows_out = n_tok * seg
    per_w = rows_out // nw
    k6 = per_w // _LN

    mesh = plsc.VectorSubcoreMesh(core_axis_name="c", subcore_axis_name="s")

    @functools.partial(
        pl.kernel,
        mesh=mesh,
        out_type=jax.ShapeDtypeStruct((rows_out, _LN), jnp.float32),
        scratch_types=[
            pltpu.VMEM((k6, _LN), jnp.int32),
            pltpu.VMEM((per_w, _LN), jnp.float32),
            pltpu.SemaphoreType.DMA,
        ],
    )
    def gather_kernel(idx_hbm, table_hbm, out_hbm, idx_v, rows_v, sem):
        wid = lax.axis_index("s") * nc + lax.axis_index("c")
        base = wid * per_w
        for b in range(k6):
            pltpu.sync_copy(idx_hbm.at[pl.ds(base + b * _LN, _LN)],
                            idx_v.at[b])
        copies = [
            pltpu.async_copy(table_hbm.at[idx_v.at[b]],
                             rows_v.at[pl.ds(b * _LN, _LN)], sem)
            for b in range(k6)
        ]
        for cp in copies:
            cp.wait()
        pltpu.sync_copy(rows_v, out_hbm.at[pl.ds(base, per_w)])

    return gather_kernel(idx2, table2)


def _cat3(w, axis=0):
    hi, lo = _split3(w)
    return jnp.concatenate([hi, lo, hi], axis=axis)


def _dot3(a, b_cat, dn):
    a_hi = a.astype(jnp.bfloat16)
    a_lo = (a - a_hi.astype(jnp.float32)).astype(jnp.bfloat16)
    a_cat = jnp.concatenate([a_hi, a_hi, a_lo], axis=1)
    return lax.dot_general(a_cat, b_cat, dn,
                           preferred_element_type=jnp.float32)


_NT = (((1,), (0,)), ((), ()))
_TT = (((1,), (1,)), ((), ()))


_T0 = 16


def _two_level_scan(x, tp, rr, dd):
    t1 = rr // _T0
    x4 = x.reshape(t1, _T0, dd)
    s = 1
    while s < _T0:
        dk = tp[s - 1:s, :].reshape(1, 1, dd)
        zpad = jnp.zeros((t1, s, dd), jnp.float32)
        x4 = x4 + dk * jnp.concatenate([zpad, x4[:, :_T0 - s, :]], axis=1)
        s *= 2
    blk = x4[:, _T0 - 1, :]
    inc = blk
    s = 1
    while s < t1:
        dk = tp[s * _T0 - 1:s * _T0, :]
        zpad = jnp.zeros((s, dd), jnp.float32)
        inc = inc + dk * jnp.concatenate([zpad, inc[:t1 - s, :]], axis=0)
        s *= 2
    exc = jnp.concatenate([jnp.zeros((1, dd), jnp.float32), inc[:t1 - 1, :]],
                          axis=0)
    dsub = tp[:_T0, :].reshape(1, _T0, dd)
    x4 = x4 + dsub * exc.reshape(t1, 1, dd)
    return x4.reshape(rr, dd)


def _mega_body(cpb, cap, h_ref, td_ref, wrkv_ref, wo_ref, mk_ref, mv_ref,
               wc_ref, wd_ref, wout_ref, bout_ref,
               out_ref, cn_ref, cd_ref, a_ref, b_ref):
    i = pl.program_id(0)
    seg = h_ref.shape[1]
    rr = h_ref.shape[0] * h_ref.shape[2]
    dd = seg * h_ref.shape[3]

    @pl.when(i % cpb == 0)
    def _():
        cn_ref[...] = jnp.zeros_like(cn_ref)
        cd_ref[...] = jnp.zeros_like(cd_ref)

    h4 = h_ref[...]
    h = jnp.concatenate(
        [h4[:, b, :, :].reshape(rr, _LN) for b in range(seg)], axis=1)
    decay = jnp.exp(-jnp.exp(td_ref[...]))

    rkv = lax.dot_general(h, wrkv_ref[...], _NT,
                          preferred_element_type=jnp.float32)
    r = jax.nn.sigmoid(rkv[:, :dd])
    k = rkv[:, dd:2 * dd]
    v = rkv[:, 2 * dd:]
    ek = jnp.exp(jnp.clip(k, -30.0, 30.0))

    a_ref[...] = ek * v
    b_ref[...] = ek

    unroll = 16

    def _step(t, carry):
        st_n, st_d = carry
        base = t * unroll
        for j in range(unroll):
            st_n = decay * st_n + a_ref[pl.ds(base + j, 1), :]
            st_d = decay * st_d + b_ref[pl.ds(base + j, 1), :]
            a_ref[pl.ds(base + j, 1), :] = st_n
            b_ref[pl.ds(base + j, 1), :] = st_d
        return st_n, st_d

    st = lax.fori_loop(0, rr // unroll, _step,
                       (cn_ref[...], cd_ref[...]))
    cn_ref[...] = st[0]
    cd_ref[...] = st[1]
    wkv = a_ref[...] / (b_ref[...] + 1e-6)

    h2 = h + lax.dot_general(r * wkv, wo_ref[...], _NT,
                             preferred_element_type=jnp.float32)

    q = lax.dot_general(h2, wc_ref[...], _NT,
                        preferred_element_type=jnp.float32)
    c_dim = q.shape[1]
    scores = lax.dot_general(q, mk_ref[...], _TT,
                             preferred_element_type=jnp.float32)
    scores = scores * jnp.float32(1.0 / (c_dim ** 0.5))

    col = lax.broadcasted_iota(jnp.int32, (rr, cap), 1)
    m1 = jnp.max(scores, axis=1, keepdims=True)
    i1 = jnp.min(jnp.where(scores == m1, col, cap), axis=1, keepdims=True)
    masked = jnp.where(col == i1, jnp.float32(-jnp.inf), scores)
    m2 = jnp.max(masked, axis=1, keepdims=True)
    i2 = jnp.min(jnp.where(masked == m2, col, cap), axis=1, keepdims=True)
    e2 = jnp.exp(m2 - m1)
    w1 = 1.0 / (1.0 + e2)
    w2 = e2 / (1.0 + e2)
    wsel = jnp.where(col == i1, w1, 0.0) + jnp.where(col == i2, w2, 0.0)

    read = jnp.dot(wsel, mv_ref[...], preferred_element_type=jnp.float32)
    h3 = h2 + jnp.dot(read.astype(jnp.bfloat16), wd_ref[...],
                      preferred_element_type=jnp.float32)
    outt = lax.dot_general(wout_ref[...], h3.astype(jnp.bfloat16),
                           (((1,), (1,)), ((), ())),
                           preferred_element_type=jnp.float32)
    out_ref[...] = (outt + bout_ref[...]).reshape(1, outt.shape[0],
                                                  outt.shape[1])


def _tc_forward(h4, d, td, wr, wk, wv, wo, mem_keys, mem_values, wc, wd,
                w_out, b_out, chunks_per_batch):
    seg = d // _LN
    n_tok = h4.shape[0] // seg
    cap, c = mem_keys.shape
    vocab = w_out.shape[1]
    n_chunks = n_tok // _R
    h4 = h4.reshape(n_tok // 8, seg, 8, _LN)

    fixed = lambda i: (0, 0)
    call = pl.pallas_call(
        functools.partial(_mega_body, chunks_per_batch, cap),
        grid=(n_chunks,),
        in_specs=[
            pl.BlockSpec((_R // 8, seg, 8, _LN), lambda i: (i, 0, 0, 0)),
            pl.BlockSpec((1, d), fixed),
            pl.BlockSpec((d, 3 * d), fixed),
            pl.BlockSpec((d, d), fixed),
            pl.BlockSpec((cap, c), fixed),
            pl.BlockSpec((cap, c), fixed),
            pl.BlockSpec((d, c), fixed),
            pl.BlockSpec((c, d), fixed),
            pl.BlockSpec((vocab, d), fixed),
            pl.BlockSpec((vocab, 1), fixed),
        ],
        out_specs=pl.BlockSpec(
            (1, vocab, _R),
            lambda i: (i // chunks_per_batch, 0, i % chunks_per_batch)),
        out_shape=jax.ShapeDtypeStruct(
            (n_tok // (chunks_per_batch * _R), vocab,
             chunks_per_batch * _R), jnp.float32),
        scratch_shapes=[
            pltpu.VMEM((1, d), jnp.float32),
            pltpu.VMEM((1, d), jnp.float32),
            pltpu.VMEM((_R, d), jnp.float32),
            pltpu.VMEM((_R, d), jnp.float32),
        ],
        compiler_params=pltpu.CompilerParams(
            dimension_semantics=("arbitrary",),
        ),
    )
    wrkv = jnp.concatenate([wr, wk, wv], axis=1)
    return call(h4, td, wrkv, wo, mem_keys, mem_values, wc,
                wd.astype(jnp.bfloat16), w_out.T.astype(jnp.bfloat16),
                b_out.reshape(-1, 1))


def kernel(x, embed_table, time_decay, Wr, Wk, Wv, Wo, mem_keys, mem_values,
           Wc, Wd, W_out, b_out):
    b, s = x.shape
    vocab = W_out.shape[1]
    d = embed_table.shape[1]
    x_flat = x.reshape(-1).astype(jnp.int32)
    h4 = _embed_gather(x_flat, embed_table)
    out = _tc_forward(h4, d, time_decay.reshape(1, -1), Wr, Wk, Wv, Wo,
                      mem_keys, mem_values, Wc, Wd, W_out,
                      b_out, chunks_per_batch=s // _R)
    return out.swapaxes(1, 2)


# speedup vs baseline: 1.2296x; 1.0035x over previous
import functools

import jax
import jax.numpy as jnp
from jax import lax
from jax.experimental import pallas as pl
from jax.experimental.pallas import tpu as pltpu
from jax.experimental.pallas import tpu_sc as plsc

_R = 1024


_LN = 128


def _embed_gather(x_flat, table):
    n_tok = x_flat.shape[0]
    d = table.shape[1]
    seg = d // _LN
    idx2 = (x_flat.reshape(-1, 8)[:, None, :] * seg
            + jnp.arange(seg, dtype=jnp.int32)[None, :, None]).reshape(-1)
    table2 = table.reshape(-1, _LN)

    info = plsc.get_sparse_core_info()
    nc, ns = info.num_cores, info.num_subcores
    nw = nc * ns
    rows_out = n_tok * seg
    per_w = rows_out // nw
    k6 = per_w // _LN

    mesh = plsc.VectorSubcoreMesh(core_axis_name="c", subcore_axis_name="s")

    @functools.partial(
        pl.kernel,
        mesh=mesh,
        out_type=jax.ShapeDtypeStruct((rows_out, _LN), jnp.float32),
        scratch_types=[
            pltpu.VMEM((k6, _LN), jnp.int32),
            pltpu.VMEM((per_w, _LN), jnp.float32),
            pltpu.SemaphoreType.DMA,
        ],
    )
    def gather_kernel(idx_hbm, table_hbm, out_hbm, idx_v, rows_v, sem):
        wid = lax.axis_index("s") * nc + lax.axis_index("c")
        base = wid * per_w
        for b in range(k6):
            pltpu.sync_copy(idx_hbm.at[pl.ds(base + b * _LN, _LN)],
                            idx_v.at[b])
        copies = [
            pltpu.async_copy(table_hbm.at[idx_v.at[b]],
                             rows_v.at[pl.ds(b * _LN, _LN)], sem)
            for b in range(k6)
        ]
        for cp in copies:
            cp.wait()
        pltpu.sync_copy(rows_v, out_hbm.at[pl.ds(base, per_w)])

    return gather_kernel(idx2, table2)


_NT = (((1,), (0,)), ((), ()))
_TT = (((1,), (1,)), ((), ()))


def _mega_body(cpb, cap, h_ref, td_ref, wrkv_ref, wo_ref, mk_ref, mv_ref,
               wc_ref, wd_ref, wout_ref, bout_ref,
               out_ref, cn_ref, cd_ref, a_ref, b_ref):
    i = pl.program_id(0)
    seg = h_ref.shape[1]
    rr = h_ref.shape[0] * h_ref.shape[2]
    dd = seg * h_ref.shape[3]

    @pl.when(i % cpb == 0)
    def _():
        cn_ref[...] = jnp.zeros_like(cn_ref)
        cd_ref[...] = jnp.zeros_like(cd_ref)

    h4 = h_ref[...]
    h = jnp.concatenate(
        [h4[:, b, :, :].reshape(rr, _LN) for b in range(seg)], axis=1)
    decay = jnp.exp(-jnp.exp(td_ref[...]))

    rkv = lax.dot_general(h, wrkv_ref[...], _NT,
                          preferred_element_type=jnp.float32)
    r = jax.nn.sigmoid(rkv[:, :dd])
    k = rkv[:, dd:2 * dd]
    v = rkv[:, 2 * dd:]
    ek = jnp.exp(jnp.clip(k, -30.0, 30.0))

    a_ref[...] = ek * v
    b_ref[...] = ek

    unroll = 16

    def _step(t, carry):
        st_n, st_d = carry
        base = t * unroll
        for j in range(unroll):
            st_n = decay * st_n + a_ref[pl.ds(base + j, 1), :]
            st_d = decay * st_d + b_ref[pl.ds(base + j, 1), :]
            a_ref[pl.ds(base + j, 1), :] = st_n
            b_ref[pl.ds(base + j, 1), :] = st_d
        return st_n, st_d

    st = lax.fori_loop(0, rr // unroll, _step,
                       (cn_ref[...], cd_ref[...]))
    cn_ref[...] = st[0]
    cd_ref[...] = st[1]
    wkv = a_ref[...] / (b_ref[...] + 1e-6)

    h2 = h + lax.dot_general(r * wkv, wo_ref[...], _NT,
                             preferred_element_type=jnp.float32)

    q = lax.dot_general(h2, wc_ref[...], _NT,
                        preferred_element_type=jnp.float32)
    c_dim = q.shape[1]
    scores = lax.dot_general(q, mk_ref[...], _TT,
                             preferred_element_type=jnp.float32)
    scores = scores * jnp.float32(1.0 / (c_dim ** 0.5))

    col = lax.broadcasted_iota(jnp.int32, (rr, cap), 1)
    m1 = jnp.max(scores, axis=1, keepdims=True)
    i1 = jnp.min(jnp.where(scores == m1, col, cap), axis=1, keepdims=True)
    masked = jnp.where(col == i1, jnp.float32(-jnp.inf), scores)
    m2 = jnp.max(masked, axis=1, keepdims=True)
    i2 = jnp.min(jnp.where(masked == m2, col, cap), axis=1, keepdims=True)
    e2 = jnp.exp(m2 - m1)
    w1 = 1.0 / (1.0 + e2)
    w2 = e2 / (1.0 + e2)
    wsel = jnp.where(col == i1, w1, 0.0) + jnp.where(col == i2, w2, 0.0)

    read = jnp.dot(wsel, mv_ref[...], preferred_element_type=jnp.float32)
    h3 = h2 + jnp.dot(read.astype(jnp.bfloat16), wd_ref[...],
                      preferred_element_type=jnp.float32)
    outt = lax.dot_general(wout_ref[...], h3.astype(jnp.bfloat16),
                           (((1,), (1,)), ((), ())),
                           preferred_element_type=jnp.float32)
    out_ref[...] = (outt + bout_ref[...]).reshape(1, outt.shape[0],
                                                  outt.shape[1])


def _tc_forward(h4, d, td, wr, wk, wv, wo, mem_keys, mem_values, wc, wd,
                w_out, b_out, chunks_per_batch):
    seg = d // _LN
    n_tok = h4.shape[0] // seg
    cap, c = mem_keys.shape
    vocab = w_out.shape[1]
    n_chunks = n_tok // _R
    h4 = h4.reshape(n_tok // 8, seg, 8, _LN)

    fixed = lambda i: (0, 0)
    call = pl.pallas_call(
        functools.partial(_mega_body, chunks_per_batch, cap),
        grid=(n_chunks,),
        in_specs=[
            pl.BlockSpec((_R // 8, seg, 8, _LN), lambda i: (i, 0, 0, 0)),
            pl.BlockSpec((1, d), fixed),
            pl.BlockSpec((d, 3 * d), fixed),
            pl.BlockSpec((d, d), fixed),
            pl.BlockSpec((cap, c), fixed),
            pl.BlockSpec((cap, c), fixed),
            pl.BlockSpec((d, c), fixed),
            pl.BlockSpec((c, d), fixed),
            pl.BlockSpec((vocab, d), fixed),
            pl.BlockSpec((vocab, 1), fixed),
        ],
        out_specs=pl.BlockSpec(
            (1, vocab, _R),
            lambda i: (i // chunks_per_batch, 0, i % chunks_per_batch)),
        out_shape=jax.ShapeDtypeStruct(
            (n_tok // (chunks_per_batch * _R), vocab,
             chunks_per_batch * _R), jnp.float32),
        scratch_shapes=[
            pltpu.VMEM((1, d), jnp.float32),
            pltpu.VMEM((1, d), jnp.float32),
            pltpu.VMEM((_R, d), jnp.float32),
            pltpu.VMEM((_R, d), jnp.float32),
        ],
        compiler_params=pltpu.CompilerParams(
            dimension_semantics=("arbitrary",),
        ),
    )
    wrkv = jnp.concatenate([wr, wk, wv], axis=1)
    return call(h4, td, wrkv, wo, mem_keys, mem_values, wc,
                wd.astype(jnp.bfloat16), w_out.T.astype(jnp.bfloat16),
                b_out.reshape(-1, 1))


def kernel(x, embed_table, time_decay, Wr, Wk, Wv, Wo, mem_keys, mem_values,
           Wc, Wd, W_out, b_out):
    b, s = x.shape
    vocab = W_out.shape[1]
    d = embed_table.shape[1]
    x_flat = x.reshape(-1).astype(jnp.int32)
    h4 = _embed_gather(x_flat, embed_table)
    out = _tc_forward(h4, d, time_decay.reshape(1, -1), Wr, Wk, Wv, Wo,
                      mem_keys, mem_values, Wc, Wd, W_out,
                      b_out, chunks_per_batch=s // _R)
    return out.swapaxes(1, 2)
